# baseline (speedup 1.0000x reference)
import jax
import jax.numpy as jnp
from jax import lax
from jax.experimental import pallas as pl
from jax.experimental.pallas import tpu as pltpu
from jax.experimental.pallas import tpu_sc as plsc

_NC = 2
_NS = 16
_NW = _NC * _NS
_L = 16

_B = 16384
_CHUNK = _B // _NW
_ROWS = 4
_COLS = _CHUNK // _ROWS

_NPAD = 1024
_NROWS = 1000


def _tc_scores_body(gu_ref, gi_ref, beta_ref, u_ref, i_ref, j_ref,
                    out_ref, oi_ref, oj_ref):
    out_ref[...] = lax.dot_general(
        gu_ref[...], gi_ref[...],
        (((1,), (1,)), ((), ())),
        preferred_element_type=jnp.float32,
    ) + beta_ref[...]

    @pl.when(pl.program_id(0) == 0)
    def _():
        u = u_ref[...]
        oi_ref[...] = u * _NPAD + i_ref[...]
        oj_ref[...] = u * _NPAD + j_ref[...]


def _sc_gather_body(g_ref, i_ref, j_ref, out_ref,
                    i_v, j_v, gi_v, gj_v, z_v, sem):
    wid = lax.axis_index("s") * _NC + lax.axis_index("c")
    cps = [pltpu.async_copy(i_ref.at[wid], i_v, sem),
           pltpu.async_copy(j_ref.at[wid], j_v, sem)]
    for c in cps:
        c.wait()
    cps = []
    for r in range(_ROWS):
        cps.append(pltpu.async_copy(g_ref.at[i_v.at[r]], gi_v.at[r], sem))
        cps.append(pltpu.async_copy(g_ref.at[j_v.at[r]], gj_v.at[r], sem))
    for c in cps:
        c.wait()
    for r in range(_ROWS):
        for t in range(_COLS // _L):
            sl = pl.ds(t * _L, _L)
            z_v[r, sl] = gi_v[r, sl] - gj_v[r, sl]
    pltpu.sync_copy(z_v, out_ref.at[wid])


def _tc_loss_body(z_ref, out_ref):
    z = z_ref[...]
    out_ref[0, 0] = -jnp.mean(jnp.log(jax.nn.sigmoid(z)))


def kernel(sampleU, sampleI, sampleJ, betaI, gammaU, gammaI):
    beta = betaI.reshape(1, _NROWS)
    u2 = sampleU.reshape(_B // 128, 128)
    i2 = sampleI.reshape(_B // 128, 128)
    j2 = sampleJ.reshape(_B // 128, 128)

    scores, oi, oj = pl.pallas_call(
        _tc_scores_body,
        grid=(8,),
        in_specs=[
            pl.BlockSpec((_NROWS, 64), lambda g: (0, 0)),
            pl.BlockSpec((128, 64), lambda g: (g, 0)),
            pl.BlockSpec((1, 128), lambda g: (0, g)),
            pl.BlockSpec((_B // 128, 128), lambda g: (0, 0)),
            pl.BlockSpec((_B // 128, 128), lambda g: (0, 0)),
            pl.BlockSpec((_B // 128, 128), lambda g: (0, 0)),
        ],
        out_specs=(
            pl.BlockSpec((_NROWS, 128), lambda g: (0, g)),
            pl.BlockSpec((_B // 128, 128), lambda g: (0, 0)),
            pl.BlockSpec((_B // 128, 128), lambda g: (0, 0)),
        ),
        out_shape=(jax.ShapeDtypeStruct((_NROWS, _NPAD), jnp.float32),
                   jax.ShapeDtypeStruct((_B // 128, 128), jnp.int32),
                   jax.ShapeDtypeStruct((_B // 128, 128), jnp.int32)),
    )(gammaU, gammaI, beta, u2, i2, j2)

    g_flat = scores.reshape(_NROWS * _NPAD)
    i3 = oi.reshape(_NW, _ROWS, _COLS)
    j3 = oj.reshape(_NW, _ROWS, _COLS)

    sc_gather = pl.kernel(
        _sc_gather_body,
        out_type=jax.ShapeDtypeStruct((_NW, _ROWS, _COLS), jnp.float32),
        mesh=plsc.VectorSubcoreMesh(core_axis_name="c", subcore_axis_name="s",
                                    num_cores=_NC, num_subcores=_NS),
        scratch_types=[
            pltpu.VMEM((_ROWS, _COLS), jnp.int32),
            pltpu.VMEM((_ROWS, _COLS), jnp.int32),
            pltpu.VMEM((_ROWS, _COLS), jnp.float32),
            pltpu.VMEM((_ROWS, _COLS), jnp.float32),
            pltpu.VMEM((_ROWS, _COLS), jnp.float32),
            pltpu.SemaphoreType.DMA,
        ],
    )
    z3 = sc_gather(g_flat, i3, j3)

    z = z3.reshape(_B // 128, 128)
    loss = pl.pallas_call(
        _tc_loss_body,
        out_shape=jax.ShapeDtypeStruct((1, 1), jnp.float32),
        out_specs=pl.BlockSpec(memory_space=pltpu.SMEM),
    )(z)
    return loss[0, 0]


# speedup vs baseline: 1.0021x; 1.0021x over previous
import jax
import jax.numpy as jnp
from jax import lax
from jax.experimental import pallas as pl
from jax.experimental.pallas import tpu as pltpu
from jax.experimental.pallas import tpu_sc as plsc

_NC = 2
_NS = 16
_NW = _NC * _NS
_L = 16

_B = 16384
_CHUNK = _B // _NW
_ROWS = 4
_COLS = _CHUNK // _ROWS

_NPAD = 1024
_NROWS = 1000


def _tc_scores_body(gu_ref, gi_ref, beta_ref, u_ref, i_ref, j_ref,
                    out_ref, oi_ref, oj_ref):
    out_ref[:, : _NROWS] = lax.dot_general(
        gu_ref[...], gi_ref[...],
        (((1,), (1,)), ((), ())),
        preferred_element_type=jnp.float32,
    ) + beta_ref[...]

    @pl.when(pl.program_id(0) == 0)
    def _():
        u = u_ref[...]
        oi_ref[...] = u * _NPAD + i_ref[...]
        oj_ref[...] = u * _NPAD + j_ref[...]


def _sc_gather_body(g_ref, i_ref, j_ref, out_ref,
                    i_v, j_v, gi_v, gj_v, z_v, sem):
    wid = lax.axis_index("s") * _NC + lax.axis_index("c")
    cps = [pltpu.async_copy(i_ref.at[wid], i_v, sem),
           pltpu.async_copy(j_ref.at[wid], j_v, sem)]
    for c in cps:
        c.wait()
    cps = []
    for r in range(_ROWS):
        cps.append(pltpu.async_copy(g_ref.at[i_v.at[r]], gi_v.at[r], sem))
        cps.append(pltpu.async_copy(g_ref.at[j_v.at[r]], gj_v.at[r], sem))
    for c in cps:
        c.wait()
    for r in range(_ROWS):
        for t in range(_COLS // _L):
            sl = pl.ds(t * _L, _L)
            z_v[r, sl] = gi_v[r, sl] - gj_v[r, sl]
    pltpu.sync_copy(z_v, out_ref.at[wid])


def _tc_loss_body(z_ref, out_ref):
    z = z_ref[...]
    out_ref[0, 0] = -jnp.mean(jnp.log(jax.nn.sigmoid(z)))


def kernel(sampleU, sampleI, sampleJ, betaI, gammaU, gammaI):
    beta = betaI.reshape(1, _NROWS)
    u2 = sampleU.reshape(_B // 128, 128)
    i2 = sampleI.reshape(_B // 128, 128)
    j2 = sampleJ.reshape(_B // 128, 128)

    scores, oi, oj = pl.pallas_call(
        _tc_scores_body,
        grid=(8,),
        in_specs=[
            pl.BlockSpec((128, 64), lambda g: (g, 0)),
            pl.BlockSpec((_NROWS, 64), lambda g: (0, 0)),
            pl.BlockSpec((1, _NROWS), lambda g: (0, 0)),
            pl.BlockSpec((_B // 128, 128), lambda g: (0, 0)),
            pl.BlockSpec((_B // 128, 128), lambda g: (0, 0)),
            pl.BlockSpec((_B // 128, 128), lambda g: (0, 0)),
        ],
        out_specs=(
            pl.BlockSpec((128, _NPAD), lambda g: (g, 0)),
            pl.BlockSpec((_B // 128, 128), lambda g: (0, 0)),
            pl.BlockSpec((_B // 128, 128), lambda g: (0, 0)),
        ),
        out_shape=(jax.ShapeDtypeStruct((_NROWS, _NPAD), jnp.float32),
                   jax.ShapeDtypeStruct((_B // 128, 128), jnp.int32),
                   jax.ShapeDtypeStruct((_B // 128, 128), jnp.int32)),
    )(gammaU, gammaI, beta, u2, i2, j2)

    g_flat = scores.reshape(_NROWS * _NPAD)
    i3 = oi.reshape(_NW, _ROWS, _COLS)
    j3 = oj.reshape(_NW, _ROWS, _COLS)

    sc_gather = pl.kernel(
        _sc_gather_body,
        out_type=jax.ShapeDtypeStruct((_NW, _ROWS, _COLS), jnp.float32),
        mesh=plsc.VectorSubcoreMesh(core_axis_name="c", subcore_axis_name="s",
                                    num_cores=_NC, num_subcores=_NS),
        scratch_types=[
            pltpu.VMEM((_ROWS, _COLS), jnp.int32),
            pltpu.VMEM((_ROWS, _COLS), jnp.int32),
            pltpu.VMEM((_ROWS, _COLS), jnp.float32),
            pltpu.VMEM((_ROWS, _COLS), jnp.float32),
            pltpu.VMEM((_ROWS, _COLS), jnp.float32),
            pltpu.SemaphoreType.DMA,
        ],
    )
    z3 = sc_gather(g_flat, i3, j3)

    z = z3.reshape(_B // 128, 128)
    loss = pl.pallas_call(
        _tc_loss_body,
        out_shape=jax.ShapeDtypeStruct((1, 1), jnp.float32),
        out_specs=pl.BlockSpec(memory_space=pltpu.SMEM),
    )(z)
    return loss[0, 0]


# speedup vs baseline: 1.0917x; 1.0894x over previous
import jax
import jax.numpy as jnp
from jax import lax
from jax.experimental import pallas as pl
from jax.experimental.pallas import tpu as pltpu
from jax.experimental.pallas import tpu_sc as plsc

_NC = 2
_NS = 16
_NW = _NC * _NS
_L = 16

_B = 16384
_CHUNK = _B // _NW
_ROWS = 4
_COLS = _CHUNK // _ROWS

_NPAD = 1024
_NROWS = 1000


def _tc_scores_body(gu_ref, gi_ref, beta_ref, u_ref, i_ref, j_ref,
                    out_ref, oi_ref, oj_ref):
    out_ref[:, : _NROWS] = lax.dot_general(
        gu_ref[...], gi_ref[...],
        (((1,), (1,)), ((), ())),
        preferred_element_type=jnp.float32,
    ) + beta_ref[...]
    u = u_ref[...]
    oi_ref[...] = u * _NPAD + i_ref[...]
    oj_ref[...] = u * _NPAD + j_ref[...]


def _sc_gather_body(g_ref, i_ref, j_ref, out_ref,
                    i_v, j_v, gi_v, gj_v, z_v, sem):
    wid = lax.axis_index("s") * _NC + lax.axis_index("c")
    cps = [pltpu.async_copy(i_ref.at[wid], i_v, sem),
           pltpu.async_copy(j_ref.at[wid], j_v, sem)]
    for c in cps:
        c.wait()
    cps = []
    for r in range(_ROWS):
        cps.append(pltpu.async_copy(g_ref.at[i_v.at[r]], gi_v.at[r], sem))
        cps.append(pltpu.async_copy(g_ref.at[j_v.at[r]], gj_v.at[r], sem))
    for c in cps:
        c.wait()
    for r in range(_ROWS):
        for t in range(_COLS // _L):
            sl = pl.ds(t * _L, _L)
            z_v[r, sl] = gi_v[r, sl] - gj_v[r, sl]
    pltpu.sync_copy(z_v, out_ref.at[wid])


def _tc_loss_body(z_ref, out_ref):
    z = z_ref[...]
    out_ref[0, 0] = -jnp.mean(jnp.log(jax.nn.sigmoid(z)))


def kernel(sampleU, sampleI, sampleJ, betaI, gammaU, gammaI):
    beta = betaI.reshape(1, _NROWS)
    u2 = sampleU.reshape(_B // 128, 128)
    i2 = sampleI.reshape(_B // 128, 128)
    j2 = sampleJ.reshape(_B // 128, 128)

    scores, oi, oj = pl.pallas_call(
        _tc_scores_body,
        out_shape=(jax.ShapeDtypeStruct((_NROWS, _NPAD), jnp.float32),
                   jax.ShapeDtypeStruct((_B // 128, 128), jnp.int32),
                   jax.ShapeDtypeStruct((_B // 128, 128), jnp.int32)),
    )(gammaU, gammaI, beta, u2, i2, j2)

    g_flat = scores.reshape(_NROWS * _NPAD)
    i3 = oi.reshape(_NW, _ROWS, _COLS)
    j3 = oj.reshape(_NW, _ROWS, _COLS)

    sc_gather = pl.kernel(
        _sc_gather_body,
        out_type=jax.ShapeDtypeStruct((_NW, _ROWS, _COLS), jnp.float32),
        mesh=plsc.VectorSubcoreMesh(core_axis_name="c", subcore_axis_name="s",
                                    num_cores=_NC, num_subcores=_NS),
        scratch_types=[
            pltpu.VMEM((_ROWS, _COLS), jnp.int32),
            pltpu.VMEM((_ROWS, _COLS), jnp.int32),
            pltpu.VMEM((_ROWS, _COLS), jnp.float32),
            pltpu.VMEM((_ROWS, _COLS), jnp.float32),
            pltpu.VMEM((_ROWS, _COLS), jnp.float32),
            pltpu.SemaphoreType.DMA,
        ],
    )
    z3 = sc_gather(g_flat, i3, j3)

    z = z3.reshape(_B // 128, 128)
    loss = pl.pallas_call(
        _tc_loss_body,
        out_shape=jax.ShapeDtypeStruct((1, 1), jnp.float32),
        out_specs=pl.BlockSpec(memory_space=pltpu.SMEM),
    )(z)
    return loss[0, 0]


# speedup vs baseline: 1.1056x; 1.0127x over previous
import jax
import jax.numpy as jnp
from jax import lax
from jax.experimental import pallas as pl
from jax.experimental.pallas import tpu as pltpu
from jax.experimental.pallas import tpu_sc as plsc

_NC = 2
_NS = 16
_NW = _NC * _NS
_L = 16

_B = 16384
_CHUNK = _B // _NW
_ROWS = 4
_COLS = _CHUNK // _ROWS

_NPAD = 1024
_NROWS = 1000


def _tc_scores_body(gu_ref, gi_ref, beta_ref, u_ref, i_ref, j_ref,
                    out_ref, oi_ref, oj_ref):
    out_ref[:, : _NROWS] = lax.dot_general(
        gu_ref[...], gi_ref[...],
        (((1,), (1,)), ((), ())),
        preferred_element_type=jnp.float32,
    ) + beta_ref[...]
    u = u_ref[...]
    oi_ref[...] = u * _NPAD + i_ref[...]
    oj_ref[...] = u * _NPAD + j_ref[...]


def _sc_gather_body(g_ref, i_ref, j_ref, out_ref,
                    i_v, j_v, gi_v, gj_v, z_v, sem):
    wid = lax.axis_index("s") * _NC + lax.axis_index("c")
    cps = [pltpu.async_copy(i_ref.at[wid], i_v, sem),
           pltpu.async_copy(j_ref.at[wid], j_v, sem)]
    for c in cps:
        c.wait()
    cps = []
    for r in range(_ROWS):
        cps.append(pltpu.async_copy(g_ref.at[i_v.at[r]], gi_v.at[r], sem))
        cps.append(pltpu.async_copy(g_ref.at[j_v.at[r]], gj_v.at[r], sem))
    for c in cps:
        c.wait()
    for r in range(_ROWS):
        for t in range(_COLS // _L):
            sl = pl.ds(t * _L, _L)
            z_v[r, sl] = gi_v[r, sl] - gj_v[r, sl]
    pltpu.sync_copy(z_v, out_ref.at[wid])


def _tc_loss_body(z_ref, out_ref):
    z = z_ref[...]
    out_ref[0, 0] = -jnp.mean(jnp.log(jax.nn.sigmoid(z)))


def kernel(sampleU, sampleI, sampleJ, betaI, gammaU, gammaI):
    beta = betaI.reshape(1, _NROWS)
    gammaU = gammaU.astype(jnp.bfloat16)
    gammaI = gammaI.astype(jnp.bfloat16)
    u2 = sampleU.reshape(_B // 128, 128)
    i2 = sampleI.reshape(_B // 128, 128)
    j2 = sampleJ.reshape(_B // 128, 128)

    scores, oi, oj = pl.pallas_call(
        _tc_scores_body,
        out_shape=(jax.ShapeDtypeStruct((_NROWS, _NPAD), jnp.float32),
                   jax.ShapeDtypeStruct((_B // 128, 128), jnp.int32),
                   jax.ShapeDtypeStruct((_B // 128, 128), jnp.int32)),
    )(gammaU, gammaI, beta, u2, i2, j2)

    g_flat = scores.reshape(_NROWS * _NPAD)
    i3 = oi.reshape(_NW, _ROWS, _COLS)
    j3 = oj.reshape(_NW, _ROWS, _COLS)

    sc_gather = pl.kernel(
        _sc_gather_body,
        out_type=jax.ShapeDtypeStruct((_NW, _ROWS, _COLS), jnp.float32),
        mesh=plsc.VectorSubcoreMesh(core_axis_name="c", subcore_axis_name="s",
                                    num_cores=_NC, num_subcores=_NS),
        scratch_types=[
            pltpu.VMEM((_ROWS, _COLS), jnp.int32),
            pltpu.VMEM((_ROWS, _COLS), jnp.int32),
            pltpu.VMEM((_ROWS, _COLS), jnp.float32),
            pltpu.VMEM((_ROWS, _COLS), jnp.float32),
            pltpu.VMEM((_ROWS, _COLS), jnp.float32),
            pltpu.SemaphoreType.DMA,
        ],
    )
    z3 = sc_gather(g_flat, i3, j3)

    z = z3.reshape(_B // 128, 128)
    loss = pl.pallas_call(
        _tc_loss_body,
        out_shape=jax.ShapeDtypeStruct((1, 1), jnp.float32),
        out_specs=pl.BlockSpec(memory_space=pltpu.SMEM),
    )(z)
    return loss[0, 0]


# speedup vs baseline: 1.1093x; 1.0034x over previous
import jax
import jax.numpy as jnp
from jax import lax
from jax.experimental import pallas as pl
from jax.experimental.pallas import tpu as pltpu
from jax.experimental.pallas import tpu_sc as plsc

_NC = 1
_NS = 16
_NW = _NC * _NS
_L = 16

_B = 16384
_CHUNK = _B // _NW
_COLS = 128
_ROWS = _CHUNK // _COLS

_NPAD = 1024
_NROWS = 1000


def _tc_scores_body(gu_ref, gi_ref, beta_ref, u_ref, i_ref, j_ref,
                    out_ref, oi_ref, oj_ref):
    out_ref[:, : _NROWS] = lax.dot_general(
        gu_ref[...], gi_ref[...],
        (((1,), (1,)), ((), ())),
        preferred_element_type=jnp.float32,
    ) + beta_ref[...]
    u = u_ref[...]
    oi_ref[...] = u * _NPAD + i_ref[...]
    oj_ref[...] = u * _NPAD + j_ref[...]


def _sc_gather_body(g_ref, i_ref, j_ref, out_ref,
                    i_v, j_v, gi_v, gj_v, z_v, sem):
    wid = lax.axis_index("s") * _NC + lax.axis_index("c")
    cps = [pltpu.async_copy(i_ref.at[wid], i_v, sem),
           pltpu.async_copy(j_ref.at[wid], j_v, sem)]
    for c in cps:
        c.wait()
    cps = []
    for r in range(_ROWS):
        cps.append(pltpu.async_copy(g_ref.at[i_v.at[r]], gi_v.at[r], sem))
        cps.append(pltpu.async_copy(g_ref.at[j_v.at[r]], gj_v.at[r], sem))
    for c in cps:
        c.wait()
    for r in range(_ROWS):
        for t in range(_COLS // _L):
            sl = pl.ds(t * _L, _L)
            z_v[r, sl] = gi_v[r, sl] - gj_v[r, sl]
    pltpu.sync_copy(z_v, out_ref.at[wid])


def _tc_loss_body(z_ref, out_ref):
    z = z_ref[...]
    out_ref[0, 0] = -jnp.mean(jnp.log(jax.nn.sigmoid(z)))


def kernel(sampleU, sampleI, sampleJ, betaI, gammaU, gammaI):
    beta = betaI.reshape(1, _NROWS)
    gammaU = gammaU.astype(jnp.bfloat16)
    gammaI = gammaI.astype(jnp.bfloat16)
    u2 = sampleU.reshape(_B // 128, 128)
    i2 = sampleI.reshape(_B // 128, 128)
    j2 = sampleJ.reshape(_B // 128, 128)

    scores, oi, oj = pl.pallas_call(
        _tc_scores_body,
        out_shape=(jax.ShapeDtypeStruct((_NROWS, _NPAD), jnp.float32),
                   jax.ShapeDtypeStruct((_B // 128, 128), jnp.int32),
                   jax.ShapeDtypeStruct((_B // 128, 128), jnp.int32)),
    )(gammaU, gammaI, beta, u2, i2, j2)

    g_flat = scores.reshape(_NROWS * _NPAD)
    i3 = oi.reshape(_NW, _ROWS, _COLS)
    j3 = oj.reshape(_NW, _ROWS, _COLS)

    sc_gather = pl.kernel(
        _sc_gather_body,
        out_type=jax.ShapeDtypeStruct((_NW, _ROWS, _COLS), jnp.float32),
        mesh=plsc.VectorSubcoreMesh(core_axis_name="c", subcore_axis_name="s",
                                    num_cores=_NC, num_subcores=_NS),
        scratch_types=[
            pltpu.VMEM((_ROWS, _COLS), jnp.int32),
            pltpu.VMEM((_ROWS, _COLS), jnp.int32),
            pltpu.VMEM((_ROWS, _COLS), jnp.float32),
            pltpu.VMEM((_ROWS, _COLS), jnp.float32),
            pltpu.VMEM((_ROWS, _COLS), jnp.float32),
            pltpu.SemaphoreType.DMA,
        ],
    )
    z3 = sc_gather(g_flat, i3, j3)

    z = z3.reshape(_B // 128, 128)
    loss = pl.pallas_call(
        _tc_loss_body,
        out_shape=jax.ShapeDtypeStruct((1, 1), jnp.float32),
        out_specs=pl.BlockSpec(memory_space=pltpu.SMEM),
    )(z)
    return loss[0, 0]
